# baseline (speedup 1.0000x reference)
import functools

import jax
import jax.numpy as jnp
from jax import lax
from jax.experimental import pallas as pl
from jax.experimental.pallas import tpu as pltpu
from jax.experimental.pallas import tpu_sc as plsc

N = 50000
E = 800000
H = 64
NTC = 50176
EP = 802816
NACC = 51200
TEDGE = 1024
TNODE = 1024
GE = EP // TEDGE
GN = NTC // TNODE
NC, NS = 2, 16
EPT = EP // NS
CHUNKS = EPT // 128
EPW = EP // (NC * NS)
WCHUNKS = EPW // 128
ROWS_PT = NACC // NS
PR_T = EP // 4 // NS
QCH = CHUNKS * 4
IDXQ = 224

_SQRT1_2 = 0.7071067811865476


def _gelu(t):
    return t * 0.5 * (1.0 + lax.erf(t * _SQRT1_2))


def _lnk(t, g, b):
    m = jnp.mean(t, axis=-1, keepdims=True)
    v = jnp.mean((t - m) ** 2, axis=-1, keepdims=True)
    return (t - m) * lax.rsqrt(v + 1e-5) * g + b


def _dot(a, b):
    return lax.dot_general(a, b, (((1,), (0,)), ((), ())),
                           preferred_element_type=jnp.float32)


def _wspec(shape):
    nd = len(shape)
    return pl.BlockSpec(shape, lambda i: (0,) * nd)


def _gather_body(xa_hbm, xb_hbm, ia_hbm, ib_hbm, m_hbm, ia_v, ib_v, r0, r1,
                 sg0, sg1, sw0, sw1):
    c = lax.axis_index("c")
    s = lax.axis_index("s")
    w = s * NC + c
    pltpu.sync_copy(ia_hbm.at[w], ia_v)
    pltpu.sync_copy(ib_hbm.at[w], ib_v)
    base = w * EPW
    pltpu.async_copy(xa_hbm.at[ia_v.at[0]], r0, sg0)

    def step(t, carry):
        j0 = 2 * t
        j1 = 2 * t + 1
        pltpu.make_async_copy(xa_hbm.at[ia_v.at[j0]], r0, sg0).wait()
        pltpu.async_copy(xb_hbm.at[ib_v.at[j0]], r0, sg0, add=True).wait()

        @pl.when(t > 0)
        def _():
            pltpu.make_async_copy(
                r1, m_hbm.at[pl.ds(base + (j0 - 1) * 128, 128)], sw1).wait()

        pltpu.async_copy(xa_hbm.at[ia_v.at[j1]], r1, sg1)
        pltpu.async_copy(r0, m_hbm.at[pl.ds(base + j0 * 128, 128)], sw0)
        pltpu.make_async_copy(xa_hbm.at[ia_v.at[j1]], r1, sg1).wait()
        pltpu.async_copy(xb_hbm.at[ib_v.at[j1]], r1, sg1, add=True).wait()
        pltpu.make_async_copy(
            r0, m_hbm.at[pl.ds(base + j0 * 128, 128)], sw0).wait()

        @pl.when(t < WCHUNKS // 2 - 1)
        def _():
            pltpu.async_copy(xa_hbm.at[ia_v.at[j0 + 2]], r0, sg0)

        pltpu.async_copy(r1, m_hbm.at[pl.ds(base + j1 * 128, 128)], sw1)
        return carry

    lax.fori_loop(0, WCHUNKS // 2, step, 0)
    pltpu.make_async_copy(
        r1, m_hbm.at[pl.ds(base + (WCHUNKS - 1) * 128, 128)], sw1).wait()


def _scatter_body(e_hbm, qidx_hbm, z_hbm, out_hbm, idx_v, eb0, eb1, eb2, eb3,
                  acc):
    c = lax.axis_index("c")
    s = lax.axis_index("s")
    pltpu.sync_copy(z_hbm, acc.at[pl.ds(s * ROWS_PT, ROWS_PT)])
    plsc.subcore_barrier()
    prbase = s * PR_T
    ebs = (eb0, eb1, eb2, eb3)

    def outer(g, carry):
        pltpu.sync_copy(qidx_hbm.at[s, pl.ds(g * IDXQ, IDXQ)], idx_v)

        def body(kk, carry2):
            rows = prbase + (g * (IDXQ // 4) + kk) * 32
            for q in range(4):
                pltpu.sync_copy(
                    e_hbm.at[c, pl.ds(rows, 32), pl.ds(32 * q, 32)], ebs[q])
            for q in range(4):
                pltpu.sync_copy(ebs[q], acc.at[idx_v.at[kk * 4 + q]], add=True)
            return carry2

        lax.fori_loop(0, IDXQ // 4, body, carry)
        return carry

    lax.fori_loop(0, QCH // IDXQ, outer, 0)
    plsc.subcore_barrier()
    pltpu.sync_copy(acc.at[pl.ds(s * ROWS_PT, ROWS_PT)],
                    out_hbm.at[c, pl.ds(s * ROWS_PT, ROWS_PT)])


@functools.cache
def _sc_kernels():
    mesh = plsc.VectorSubcoreMesh(core_axis_name="c", subcore_axis_name="s",
                                  num_cores=NC, num_subcores=NS)
    gather = pl.kernel(
        _gather_body,
        out_type=jax.ShapeDtypeStruct((EP, 2 * H), jnp.float32),
        mesh=mesh,
        scratch_types=[
            pltpu.VMEM((WCHUNKS, 128), jnp.int32),
            pltpu.VMEM((WCHUNKS, 128), jnp.int32),
            pltpu.VMEM((128, 2 * H), jnp.float32),
            pltpu.VMEM((128, 2 * H), jnp.float32),
            pltpu.SemaphoreType.DMA,
            pltpu.SemaphoreType.DMA,
            pltpu.SemaphoreType.DMA,
            pltpu.SemaphoreType.DMA,
        ],
    )
    scatter = pl.kernel(
        _scatter_body,
        out_type=jax.ShapeDtypeStruct((2, NACC, 32), jnp.float32),
        mesh=mesh,
        scratch_types=[
            pltpu.VMEM((IDXQ, 32), jnp.int32),
            pltpu.VMEM((32, 32), jnp.float32),
            pltpu.VMEM((32, 32), jnp.float32),
            pltpu.VMEM((32, 32), jnp.float32),
            pltpu.VMEM((32, 32), jnp.float32),
            pltpu.VMEM_SHARED((NACC, 32), jnp.float32),
        ],
        compiler_params=pltpu.CompilerParams(use_tc_tiling_on_sc=False),
    )
    return gather, scatter


def _gather_sc(xa, xb, ia, ib):
    return _sc_kernels()[0](xa, xb, ia, ib)


def _scatter_sc(e2, dst_s, zrows):
    return _sc_kernels()[1](e2, dst_s, zrows)


def _input_mlp(x8, w1, b1, g1, e1, w2, b2, g2, e2):
    def body(x_ref, w1r, b1r, g1r, e1r, w2r, b2r, g2r, e2r, o_ref):
        h = _lnk(_dot(x_ref[...], w1r[...]) + b1r[...], g1r[...], e1r[...])
        h = _gelu(h)
        o_ref[...] = _lnk(_dot(h, w2r[...]) + b2r[...], g2r[...], e2r[...])

    return pl.pallas_call(
        body,
        grid=(GN,),
        in_specs=[pl.BlockSpec((TNODE, 8), lambda i: (i, 0)),
                  _wspec((8, H)), _wspec((1, H)), _wspec((1, H)), _wspec((1, H)),
                  _wspec((H, H)), _wspec((1, H)), _wspec((1, H)), _wspec((1, H))],
        out_specs=pl.BlockSpec((TNODE, H), lambda i: (i, 0)),
        out_shape=jax.ShapeDtypeStruct((NTC, H), jnp.float32),
    )(x8, w1, b1, g1, e1, w2, b2, g2, e2)


def _pre_mlp(x, wa, wb, b1):
    def body(x_ref, war, wbr, b1r, oa_ref, ob_ref):
        xv = x_ref[...]
        oa_ref[...] = (_dot(xv, war[...]) + b1r[...])
        ob_ref[...] = _dot(xv, wbr[...])

    return pl.pallas_call(
        body,
        grid=(GN,),
        in_specs=[pl.BlockSpec((TNODE, H), lambda i: (i, 0)),
                  _wspec((H, 2 * H)), _wspec((H, 2 * H)), _wspec((1, 2 * H))],
        out_specs=[pl.BlockSpec((TNODE, 2 * H), lambda i: (i, 0)),
                   pl.BlockSpec((TNODE, 2 * H), lambda i: (i, 0))],
        out_shape=[jax.ShapeDtypeStruct((NTC, 2 * H), jnp.float32),
                   jax.ShapeDtypeStruct((NTC, 2 * H), jnp.float32)],
    )(x, wa, wb, b1)


def _edge_mlp(m, eap, wc, g1, e1, w2, b2, g2, e2):
    def body(m_ref, ea_ref, wcr, g1r, e1r, w2r, b2r, g2r, e2r, o_ref):
        eab = ea_ref[...]
        ea = jnp.concatenate([eab[:, 8 * k:8 * (k + 1)] for k in range(16)],
                             axis=0)
        mm = m_ref[...] + _dot(ea, wcr[...])
        e = mm[:, :H] * mm[:, H:]
        e = _gelu(_lnk(e, g1r[...], e1r[...]))
        e = _dot(e, w2r[...]) + b2r[...]
        e = _gelu(_lnk(e, g2r[...], e2r[...]))
        h0 = e[:, :32]
        h1 = e[:, 32:]
        o_ref[0] = jnp.concatenate(
            [h0[256 * q:256 * (q + 1)] for q in range(4)], axis=1)
        o_ref[1] = jnp.concatenate(
            [h1[256 * q:256 * (q + 1)] for q in range(4)], axis=1)

    return pl.pallas_call(
        body,
        grid=(GE,),
        in_specs=[pl.BlockSpec((TEDGE, 2 * H), lambda i: (i, 0)),
                  pl.BlockSpec((TEDGE // 16, 128), lambda i: (i, 0)),
                  _wspec((8, 2 * H)), _wspec((1, H)), _wspec((1, H)),
                  _wspec((H, H)), _wspec((1, H)), _wspec((1, H)), _wspec((1, H))],
        out_specs=pl.BlockSpec((2, TEDGE // 4, 128), lambda i: (0, i, 0)),
        out_shape=jax.ShapeDtypeStruct((2, EP // 4, 128), jnp.float32),
    )(m, eap, wc, g1, e1, w2, b2, g2, e2)


def _node_mlp(x, agg, wx, wa0, wa1, b1, g1, e1, w2, b2, g2, e2):
    def body(x_ref, a_ref, wxr, wa0r, wa1r, b1r, g1r, e1r, w2r, b2r, g2r, e2r,
             o_ref):
        xv = x_ref[...]
        u = (_dot(xv, wxr[...]) + _dot(a_ref[0], wa0r[...])
             + _dot(a_ref[1], wa1r[...]) + b1r[...])
        u = _gelu(_lnk(u, g1r[...], e1r[...]))
        u = _lnk(_dot(u, w2r[...]) + b2r[...], g2r[...], e2r[...])
        o_ref[...] = _gelu(xv + u)

    return pl.pallas_call(
        body,
        grid=(GN,),
        in_specs=[pl.BlockSpec((TNODE, H), lambda i: (i, 0)),
                  pl.BlockSpec((2, TNODE, 32), lambda i: (0, i, 0)),
                  _wspec((H, H)), _wspec((32, H)), _wspec((32, H)),
                  _wspec((1, H)), _wspec((1, H)), _wspec((1, H)),
                  _wspec((H, H)), _wspec((1, H)), _wspec((1, H)), _wspec((1, H))],
        out_specs=pl.BlockSpec((TNODE, H), lambda i: (i, 0)),
        out_shape=jax.ShapeDtypeStruct((NTC, H), jnp.float32),
    )(x, agg, wx, wa0, wa1, b1, g1, e1, w2, b2, g2, e2)


def _out_mlp(x, w1, b1, w2, b2):
    def body(x_ref, w1r, b1r, w2r, b2r, o_ref):
        h = _gelu(_dot(x_ref[...], w1r[...]) + b1r[...])
        o_ref[...] = _dot(h, w2r[...]) + b2r[...]

    return pl.pallas_call(
        body,
        grid=(GN,),
        in_specs=[pl.BlockSpec((TNODE, H), lambda i: (i, 0)),
                  _wspec((H, H)), _wspec((1, H)), _wspec((H, 8)), _wspec((1, 8))],
        out_specs=pl.BlockSpec((TNODE, 8), lambda i: (i, 0)),
        out_shape=jax.ShapeDtypeStruct((NTC, 8), jnp.float32),
    )(x, w1, b1, w2, b2)


def kernel(x, edge_index, edge_attr, params):
    p = params
    f32 = jnp.float32

    x8 = jnp.zeros((NTC, 8), f32).at[:N, :5].set(x)
    src = edge_index[0]
    dst = edge_index[1]
    padg = EP - E
    zpad = jnp.zeros((padg,), jnp.int32)
    ia = jnp.concatenate([dst, zpad]).reshape(NC * NS, WCHUNKS, 128)
    ib = jnp.concatenate([src, zpad]).reshape(NC * NS, WCHUNKS, 128)
    qidx = jnp.concatenate(
        [dst, jnp.full((padg,), N, jnp.int32)]).reshape(
            NS, 49, 4, 8, 32).transpose(0, 1, 3, 2, 4).reshape(NS, QCH, 32)
    ea8 = jnp.zeros((EP, 8), f32).at[:E, :4].set(edge_attr)
    eap = ea8.reshape(GE, 16, 64, 8).transpose(0, 2, 1, 3).reshape(EP // 16, 128)
    zrows = jnp.zeros((ROWS_PT, 32), f32)

    iw1 = jnp.zeros((8, H), f32).at[:5].set(p['iW1'])
    xc = _input_mlp(x8, iw1, p['ib1'][None], p['iln1_g'][None],
                    p['iln1_b'][None], p['iW2'], p['ib2'][None],
                    p['iln2_g'][None], p['iln2_b'][None])

    for l in range(3):
        xa, xb = _pre_mlp(xc, p['eW1'][l][:H], p['eW1'][l][H:2 * H],
                          p['eb1'][l][None])
        m = _gather_sc(xa, xb, ia, ib)
        wc = jnp.zeros((8, 2 * H), f32).at[:4].set(p['eW1'][l][2 * H:])
        e2 = _edge_mlp(m, eap, wc,
                       p['eln1_g'][l][None], p['eln1_b'][l][None],
                       p['eW2'][l], p['eb2'][l][None],
                       p['eln2_g'][l][None], p['eln2_b'][l][None])
        agg = _scatter_sc(e2, qidx, zrows)
        xc = _node_mlp(xc, agg,
                       p['nW1'][l][:H], p['nW1'][l][H:H + 32],
                       p['nW1'][l][H + 32:], p['nb1'][l][None],
                       p['nln1_g'][l][None], p['nln1_b'][l][None],
                       p['nW2'][l], p['nb2'][l][None],
                       p['nln2_g'][l][None], p['nln2_b'][l][None])

    ow2 = jnp.zeros((H, 8), f32).at[:, :1].set(p['oW2'])
    ob2 = jnp.zeros((1, 8), f32).at[0, 0].set(p['ob2'][0])
    out8 = _out_mlp(xc, p['oW1'], p['ob1'][None], ow2, ob2)
    return out8[:N, 0]


# speedup vs baseline: 1.0395x; 1.0395x over previous
import functools

import jax
import jax.numpy as jnp
from jax import lax
from jax.experimental import pallas as pl
from jax.experimental.pallas import tpu as pltpu
from jax.experimental.pallas import tpu_sc as plsc

N = 50000
E = 800000
H = 64
NTC = 50176
EP = 802816
NACC = 51200
TEDGE = 1024
TNODE = 1024
GE = EP // TEDGE
GN = NTC // TNODE
NC, NS = 2, 16
EPT = EP // NS
CHUNKS = EPT // 128
EPW = EP // (NC * NS)
WCHUNKS = EPW // 128
ROWS_PT = NACC // NS
IDXC = 56

_SQRT1_2 = 0.7071067811865476


def _gelu(t):
    return t * 0.5 * (1.0 + lax.erf(t * _SQRT1_2))


def _lnk(t, g, b):
    m = jnp.mean(t, axis=-1, keepdims=True)
    v = jnp.mean((t - m) ** 2, axis=-1, keepdims=True)
    return (t - m) * lax.rsqrt(v + 1e-5) * g + b


def _dot(a, b):
    return lax.dot_general(a, b, (((1,), (0,)), ((), ())),
                           preferred_element_type=jnp.float32)


def _wspec(shape):
    nd = len(shape)
    return pl.BlockSpec(shape, lambda i: (0,) * nd)


def _gather_body(xa_hbm, xb_hbm, ia_hbm, ib_hbm, m_hbm, ia_v, ib_v, rows_v,
                 sem):
    c = lax.axis_index("c")
    s = lax.axis_index("s")
    w = s * NC + c
    pltpu.sync_copy(ia_hbm.at[w], ia_v)
    pltpu.sync_copy(ib_hbm.at[w], ib_v)
    base = w * EPW

    def body(j, carry):
        pltpu.async_copy(xa_hbm.at[ia_v.at[j]], rows_v, sem).wait()
        pltpu.async_copy(xb_hbm.at[ib_v.at[j]], rows_v, sem, add=True).wait()
        pltpu.sync_copy(rows_v, m_hbm.at[pl.ds(base + j * 128, 128)])
        return carry

    lax.fori_loop(0, WCHUNKS, body, 0)


def _scatter_body(e_hbm, dst_hbm, z_hbm, out_hbm, idx_v, ebuf, acc):
    c = lax.axis_index("c")
    s = lax.axis_index("s")
    pltpu.sync_copy(z_hbm, acc.at[pl.ds(s * ROWS_PT, ROWS_PT)])
    plsc.subcore_barrier()
    base = s * EPT

    def outer(k, carry):
        pltpu.sync_copy(dst_hbm.at[s, pl.ds(k * IDXC, IDXC)], idx_v)

        def body(j, carry2):
            pltpu.sync_copy(
                e_hbm.at[c, pl.ds(base + (k * IDXC + j) * 128, 128)], ebuf)
            pltpu.sync_copy(ebuf, acc.at[idx_v.at[j]], add=True)
            return carry2

        lax.fori_loop(0, IDXC, body, carry)
        return carry

    lax.fori_loop(0, CHUNKS // IDXC, outer, 0)
    plsc.subcore_barrier()
    pltpu.sync_copy(acc.at[pl.ds(s * ROWS_PT, ROWS_PT)],
                    out_hbm.at[c, pl.ds(s * ROWS_PT, ROWS_PT)])


@functools.cache
def _sc_kernels():
    mesh = plsc.VectorSubcoreMesh(core_axis_name="c", subcore_axis_name="s",
                                  num_cores=NC, num_subcores=NS)
    gather = pl.kernel(
        _gather_body,
        out_type=jax.ShapeDtypeStruct((EP, 2 * H), jnp.float32),
        mesh=mesh,
        scratch_types=[
            pltpu.VMEM((WCHUNKS, 128), jnp.int32),
            pltpu.VMEM((WCHUNKS, 128), jnp.int32),
            pltpu.VMEM((128, 2 * H), jnp.float32),
            pltpu.SemaphoreType.DMA,
        ],
    )
    scatter = pl.kernel(
        _scatter_body,
        out_type=jax.ShapeDtypeStruct((2, NACC, 32), jnp.float32),
        mesh=mesh,
        scratch_types=[
            pltpu.VMEM((IDXC, 128), jnp.int32),
            pltpu.VMEM((128, 32), jnp.float32),
            pltpu.VMEM_SHARED((NACC, 32), jnp.float32),
        ],
        compiler_params=pltpu.CompilerParams(use_tc_tiling_on_sc=False),
    )
    return gather, scatter


def _gather_sc(xa, xb, ia, ib):
    return _sc_kernels()[0](xa, xb, ia, ib)


def _scatter_sc(e2, dst_s, zrows):
    return _sc_kernels()[1](e2, dst_s, zrows)


def _input_mlp(x8, w1, b1, g1, e1, w2, b2, g2, e2):
    def body(x_ref, w1r, b1r, g1r, e1r, w2r, b2r, g2r, e2r, o_ref):
        h = _lnk(_dot(x_ref[...], w1r[...]) + b1r[...], g1r[...], e1r[...])
        h = _gelu(h)
        o_ref[...] = _lnk(_dot(h, w2r[...]) + b2r[...], g2r[...], e2r[...])

    return pl.pallas_call(
        body,
        grid=(GN,),
        in_specs=[pl.BlockSpec((TNODE, 8), lambda i: (i, 0)),
                  _wspec((8, H)), _wspec((1, H)), _wspec((1, H)), _wspec((1, H)),
                  _wspec((H, H)), _wspec((1, H)), _wspec((1, H)), _wspec((1, H))],
        out_specs=pl.BlockSpec((TNODE, H), lambda i: (i, 0)),
        out_shape=jax.ShapeDtypeStruct((NTC, H), jnp.float32),
    )(x8, w1, b1, g1, e1, w2, b2, g2, e2)


def _pre_mlp(x, wa, wb, b1):
    def body(x_ref, war, wbr, b1r, oa_ref, ob_ref):
        xv = x_ref[...]
        oa_ref[...] = (_dot(xv, war[...]) + b1r[...])
        ob_ref[...] = _dot(xv, wbr[...])

    return pl.pallas_call(
        body,
        grid=(GN,),
        in_specs=[pl.BlockSpec((TNODE, H), lambda i: (i, 0)),
                  _wspec((H, 2 * H)), _wspec((H, 2 * H)), _wspec((1, 2 * H))],
        out_specs=[pl.BlockSpec((TNODE, 2 * H), lambda i: (i, 0)),
                   pl.BlockSpec((TNODE, 2 * H), lambda i: (i, 0))],
        out_shape=[jax.ShapeDtypeStruct((NTC, 2 * H), jnp.float32),
                   jax.ShapeDtypeStruct((NTC, 2 * H), jnp.float32)],
    )(x, wa, wb, b1)


def _edge_mlp(m, eap, wc, g1, e1, w2, b2, g2, e2):
    def body(m_ref, ea_ref, wcr, g1r, e1r, w2r, b2r, g2r, e2r, o_ref):
        eab = ea_ref[...]
        ea = jnp.concatenate([eab[:, 8 * k:8 * (k + 1)] for k in range(16)],
                             axis=0)
        mm = m_ref[...] + _dot(ea, wcr[...])
        e = mm[:, :H] * mm[:, H:]
        e = _gelu(_lnk(e, g1r[...], e1r[...]))
        e = _dot(e, w2r[...]) + b2r[...]
        e = _gelu(_lnk(e, g2r[...], e2r[...]))
        o_ref[0] = e[:, :32]
        o_ref[1] = e[:, 32:]

    return pl.pallas_call(
        body,
        grid=(GE,),
        in_specs=[pl.BlockSpec((TEDGE, 2 * H), lambda i: (i, 0)),
                  pl.BlockSpec((TEDGE // 16, 128), lambda i: (i, 0)),
                  _wspec((8, 2 * H)), _wspec((1, H)), _wspec((1, H)),
                  _wspec((H, H)), _wspec((1, H)), _wspec((1, H)), _wspec((1, H))],
        out_specs=pl.BlockSpec((2, TEDGE, 32), lambda i: (0, i, 0)),
        out_shape=jax.ShapeDtypeStruct((2, EP, 32), jnp.float32),
    )(m, eap, wc, g1, e1, w2, b2, g2, e2)


def _node_mlp(x, agg, wx, wa0, wa1, b1, g1, e1, w2, b2, g2, e2):
    def body(x_ref, a_ref, wxr, wa0r, wa1r, b1r, g1r, e1r, w2r, b2r, g2r, e2r,
             o_ref):
        xv = x_ref[...]
        u = (_dot(xv, wxr[...]) + _dot(a_ref[0], wa0r[...])
             + _dot(a_ref[1], wa1r[...]) + b1r[...])
        u = _gelu(_lnk(u, g1r[...], e1r[...]))
        u = _lnk(_dot(u, w2r[...]) + b2r[...], g2r[...], e2r[...])
        o_ref[...] = _gelu(xv + u)

    return pl.pallas_call(
        body,
        grid=(GN,),
        in_specs=[pl.BlockSpec((TNODE, H), lambda i: (i, 0)),
                  pl.BlockSpec((2, TNODE, 32), lambda i: (0, i, 0)),
                  _wspec((H, H)), _wspec((32, H)), _wspec((32, H)),
                  _wspec((1, H)), _wspec((1, H)), _wspec((1, H)),
                  _wspec((H, H)), _wspec((1, H)), _wspec((1, H)), _wspec((1, H))],
        out_specs=pl.BlockSpec((TNODE, H), lambda i: (i, 0)),
        out_shape=jax.ShapeDtypeStruct((NTC, H), jnp.float32),
    )(x, agg, wx, wa0, wa1, b1, g1, e1, w2, b2, g2, e2)


def _out_mlp(x, w1, b1, w2, b2):
    def body(x_ref, w1r, b1r, w2r, b2r, o_ref):
        h = _gelu(_dot(x_ref[...], w1r[...]) + b1r[...])
        o_ref[...] = _dot(h, w2r[...]) + b2r[...]

    return pl.pallas_call(
        body,
        grid=(GN,),
        in_specs=[pl.BlockSpec((TNODE, H), lambda i: (i, 0)),
                  _wspec((H, H)), _wspec((1, H)), _wspec((H, 8)), _wspec((1, 8))],
        out_specs=pl.BlockSpec((TNODE, 8), lambda i: (i, 0)),
        out_shape=jax.ShapeDtypeStruct((NTC, 8), jnp.float32),
    )(x, w1, b1, w2, b2)


def kernel(x, edge_index, edge_attr, params):
    p = params
    f32 = jnp.float32

    x8 = jnp.zeros((NTC, 8), f32).at[:N, :5].set(x)
    src = edge_index[0]
    dst = edge_index[1]
    padg = EP - E
    zpad = jnp.zeros((padg,), jnp.int32)
    ia = jnp.concatenate([dst, zpad]).reshape(NC * NS, WCHUNKS, 128)
    ib = jnp.concatenate([src, zpad]).reshape(NC * NS, WCHUNKS, 128)
    dst_s = jnp.concatenate(
        [dst, jnp.full((padg,), N, jnp.int32)]).reshape(NS, CHUNKS, 128)
    ea8 = jnp.zeros((EP, 8), f32).at[:E, :4].set(edge_attr)
    eap = ea8.reshape(GE, 16, 64, 8).transpose(0, 2, 1, 3).reshape(EP // 16, 128)
    zrows = jnp.zeros((ROWS_PT, 32), f32)

    iw1 = jnp.zeros((8, H), f32).at[:5].set(p['iW1'])
    xc = _input_mlp(x8, iw1, p['ib1'][None], p['iln1_g'][None],
                    p['iln1_b'][None], p['iW2'], p['ib2'][None],
                    p['iln2_g'][None], p['iln2_b'][None])

    for l in range(3):
        xa, xb = _pre_mlp(xc, p['eW1'][l][:H], p['eW1'][l][H:2 * H],
                          p['eb1'][l][None])
        m = _gather_sc(xa, xb, ia, ib)
        wc = jnp.zeros((8, 2 * H), f32).at[:4].set(p['eW1'][l][2 * H:])
        e2 = _edge_mlp(m, eap, wc,
                       p['eln1_g'][l][None], p['eln1_b'][l][None],
                       p['eW2'][l], p['eb2'][l][None],
                       p['eln2_g'][l][None], p['eln2_b'][l][None])
        agg = _scatter_sc(e2, dst_s, zrows)
        xc = _node_mlp(xc, agg,
                       p['nW1'][l][:H], p['nW1'][l][H:H + 32],
                       p['nW1'][l][H + 32:], p['nb1'][l][None],
                       p['nln1_g'][l][None], p['nln1_b'][l][None],
                       p['nW2'][l], p['nb2'][l][None],
                       p['nln2_g'][l][None], p['nln2_b'][l][None])

    ow2 = jnp.zeros((H, 8), f32).at[:, :1].set(p['oW2'])
    ob2 = jnp.zeros((1, 8), f32).at[0, 0].set(p['ob2'][0])
    out8 = _out_mlp(xc, p['oW1'], p['ob1'][None], ow2, ob2)
    return out8[:N, 0]


# speedup vs baseline: 1.1855x; 1.1405x over previous
import functools

import jax
import jax.numpy as jnp
from jax import lax
from jax.experimental import pallas as pl
from jax.experimental.pallas import tpu as pltpu
from jax.experimental.pallas import tpu_sc as plsc

N = 50000
E = 800000
H = 64
NTC = 50176
EP = 802816
NACC = 51200
TEDGE = 1024
TNODE = 1024
GE = EP // TEDGE
GN = NTC // TNODE
NC, NS = 2, 16
EH = EP // 2
GEH = EH // TEDGE
EPT = EH // NS
CHUNKS = EPT // 128
EPW = EH // (NC * NS)
WCHUNKS = EPW // 128
ROWS_PT = NACC // NS

_SQRT1_2 = 0.7071067811865476


def _gelu(t):
    return t * 0.5 * (1.0 + lax.erf(t * _SQRT1_2))


def _lnk(t, g, b):
    m = jnp.mean(t, axis=-1, keepdims=True)
    v = jnp.mean((t - m) ** 2, axis=-1, keepdims=True)
    return (t - m) * lax.rsqrt(v + 1e-5) * g + b


def _dot(a, b):
    return lax.dot_general(a, b, (((1,), (0,)), ((), ())),
                           preferred_element_type=jnp.float32)


def _wspec(shape):
    nd = len(shape)
    return pl.BlockSpec(shape, lambda i: (0,) * nd)


def _gather_body(xa_hbm, xb_hbm, ia_hbm, ib_hbm, m_hbm, ia_v, ib_v, rows_v,
                 sem):
    c = lax.axis_index("c")
    s = lax.axis_index("s")
    w = s * NC + c
    pltpu.sync_copy(ia_hbm.at[w], ia_v)
    pltpu.sync_copy(ib_hbm.at[w], ib_v)
    base = w * EPW

    def body(j, carry):
        pltpu.async_copy(xa_hbm.at[ia_v.at[j]], rows_v, sem).wait()
        pltpu.async_copy(xb_hbm.at[ib_v.at[j]], rows_v, sem, add=True).wait()
        pltpu.sync_copy(rows_v, m_hbm.at[pl.ds(base + j * 128, 128)])
        return carry

    lax.fori_loop(0, WCHUNKS, body, 0)


def _scatter_body(e_hbm, dst_hbm, z_hbm, out_hbm, idx_v, ebuf, acc):
    c = lax.axis_index("c")
    s = lax.axis_index("s")
    pltpu.sync_copy(z_hbm, acc.at[pl.ds(s * ROWS_PT, ROWS_PT)])
    plsc.subcore_barrier()
    base = s * EPT

    def outer(g, carry):
        pltpu.sync_copy(dst_hbm.at[s, g], idx_v)

        def body(j, carry2):
            pltpu.sync_copy(
                e_hbm.at[c, pl.ds(base + (g * 49 + j) * 128, 128)], ebuf)
            pltpu.sync_copy(ebuf, acc.at[idx_v.at[j]], add=True)
            return carry2

        lax.fori_loop(0, 49, body, carry)
        return carry

    lax.fori_loop(0, 4, outer, 0)
    plsc.subcore_barrier()
    pltpu.sync_copy(acc.at[pl.ds(s * ROWS_PT, ROWS_PT)],
                    out_hbm.at[c, pl.ds(s * ROWS_PT, ROWS_PT)])


@functools.cache
def _sc_kernels():
    mesh = plsc.VectorSubcoreMesh(core_axis_name="c", subcore_axis_name="s",
                                  num_cores=NC, num_subcores=NS)
    gather = pl.kernel(
        _gather_body,
        out_type=jax.ShapeDtypeStruct((EH, 2 * H), jnp.float32),
        mesh=mesh,
        scratch_types=[
            pltpu.VMEM((WCHUNKS, 128), jnp.int32),
            pltpu.VMEM((WCHUNKS, 128), jnp.int32),
            pltpu.VMEM((128, 2 * H), jnp.float32),
            pltpu.SemaphoreType.DMA,
        ],
    )
    scatter = pl.kernel(
        _scatter_body,
        out_type=jax.ShapeDtypeStruct((2, NACC, 32), jnp.float32),
        mesh=mesh,
        scratch_types=[
            pltpu.VMEM((49, 128), jnp.int32),
            pltpu.VMEM((128, 32), jnp.float32),
            pltpu.VMEM_SHARED((NACC, 32), jnp.float32),
        ],
        compiler_params=pltpu.CompilerParams(use_tc_tiling_on_sc=False),
    )
    return gather, scatter


def _gather_sc(xa, xb, ia, ib):
    return _sc_kernels()[0](xa, xb, ia, ib)


def _scatter_sc(e2, dst_s, zrows):
    return _sc_kernels()[1](e2, dst_s, zrows)


def _input_mlp(x8, w1, b1, g1, e1, w2, b2, g2, e2):
    def body(x_ref, w1r, b1r, g1r, e1r, w2r, b2r, g2r, e2r, o_ref):
        h = _lnk(_dot(x_ref[...], w1r[...]) + b1r[...], g1r[...], e1r[...])
        h = _gelu(h)
        o_ref[...] = _lnk(_dot(h, w2r[...]) + b2r[...], g2r[...], e2r[...])

    return pl.pallas_call(
        body,
        grid=(GN,),
        in_specs=[pl.BlockSpec((TNODE, 8), lambda i: (i, 0)),
                  _wspec((8, H)), _wspec((1, H)), _wspec((1, H)), _wspec((1, H)),
                  _wspec((H, H)), _wspec((1, H)), _wspec((1, H)), _wspec((1, H))],
        out_specs=pl.BlockSpec((TNODE, H), lambda i: (i, 0)),
        out_shape=jax.ShapeDtypeStruct((NTC, H), jnp.float32),
    )(x8, w1, b1, g1, e1, w2, b2, g2, e2)


def _pre_mlp(x, wa, wb, b1):
    def body(x_ref, war, wbr, b1r, oa_ref, ob_ref):
        xv = x_ref[...]
        oa_ref[...] = (_dot(xv, war[...]) + b1r[...])
        ob_ref[...] = _dot(xv, wbr[...])

    return pl.pallas_call(
        body,
        grid=(GN,),
        in_specs=[pl.BlockSpec((TNODE, H), lambda i: (i, 0)),
                  _wspec((H, 2 * H)), _wspec((H, 2 * H)), _wspec((1, 2 * H))],
        out_specs=[pl.BlockSpec((TNODE, 2 * H), lambda i: (i, 0)),
                   pl.BlockSpec((TNODE, 2 * H), lambda i: (i, 0))],
        out_shape=[jax.ShapeDtypeStruct((NTC, 2 * H), jnp.float32),
                   jax.ShapeDtypeStruct((NTC, 2 * H), jnp.float32)],
    )(x, wa, wb, b1)


def _edge_mlp(m, eap, wc, g1, e1, w2, b2, g2, e2):
    def body(m_ref, ea_ref, wcr, g1r, e1r, w2r, b2r, g2r, e2r, o_ref):
        eab = ea_ref[...]
        ea = jnp.concatenate([eab[:, 8 * k:8 * (k + 1)] for k in range(16)],
                             axis=0)
        mm = m_ref[...] + _dot(ea, wcr[...])
        e = mm[:, :H] * mm[:, H:]
        e = _gelu(_lnk(e, g1r[...], e1r[...]))
        e = _dot(e, w2r[...]) + b2r[...]
        e = _gelu(_lnk(e, g2r[...], e2r[...]))
        o_ref[0] = e[:, :32]
        o_ref[1] = e[:, 32:]

    return pl.pallas_call(
        body,
        grid=(GEH,),
        in_specs=[pl.BlockSpec((TEDGE, 2 * H), lambda i: (i, 0)),
                  pl.BlockSpec((TEDGE // 16, 128), lambda i: (i, 0)),
                  _wspec((8, 2 * H)), _wspec((1, H)), _wspec((1, H)),
                  _wspec((H, H)), _wspec((1, H)), _wspec((1, H)), _wspec((1, H))],
        out_specs=pl.BlockSpec((2, TEDGE, 32), lambda i: (0, i, 0)),
        out_shape=jax.ShapeDtypeStruct((2, EH, 32), jnp.float32),
    )(m, eap, wc, g1, e1, w2, b2, g2, e2)


def _node_mlp(x, agga, aggb, wx, wa0, wa1, b1, g1, e1, w2, b2, g2, e2):
    def body(x_ref, a_ref, b_ref, wxr, wa0r, wa1r, b1r, g1r, e1r, w2r, b2r,
             g2r, e2r, o_ref):
        xv = x_ref[...]
        a0 = a_ref[0] + b_ref[0]
        a1 = a_ref[1] + b_ref[1]
        u = (_dot(xv, wxr[...]) + _dot(a0, wa0r[...])
             + _dot(a1, wa1r[...]) + b1r[...])
        u = _gelu(_lnk(u, g1r[...], e1r[...]))
        u = _lnk(_dot(u, w2r[...]) + b2r[...], g2r[...], e2r[...])
        o_ref[...] = _gelu(xv + u)

    return pl.pallas_call(
        body,
        grid=(GN,),
        in_specs=[pl.BlockSpec((TNODE, H), lambda i: (i, 0)),
                  pl.BlockSpec((2, TNODE, 32), lambda i: (0, i, 0)),
                  pl.BlockSpec((2, TNODE, 32), lambda i: (0, i, 0)),
                  _wspec((H, H)), _wspec((32, H)), _wspec((32, H)),
                  _wspec((1, H)), _wspec((1, H)), _wspec((1, H)),
                  _wspec((H, H)), _wspec((1, H)), _wspec((1, H)), _wspec((1, H))],
        out_specs=pl.BlockSpec((TNODE, H), lambda i: (i, 0)),
        out_shape=jax.ShapeDtypeStruct((NTC, H), jnp.float32),
    )(x, agga, aggb, wx, wa0, wa1, b1, g1, e1, w2, b2, g2, e2)


def _out_mlp(x, w1, b1, w2, b2):
    def body(x_ref, w1r, b1r, w2r, b2r, o_ref):
        h = _gelu(_dot(x_ref[...], w1r[...]) + b1r[...])
        o_ref[...] = _dot(h, w2r[...]) + b2r[...]

    return pl.pallas_call(
        body,
        grid=(GN,),
        in_specs=[pl.BlockSpec((TNODE, H), lambda i: (i, 0)),
                  _wspec((H, H)), _wspec((1, H)), _wspec((H, 8)), _wspec((1, 8))],
        out_specs=pl.BlockSpec((TNODE, 8), lambda i: (i, 0)),
        out_shape=jax.ShapeDtypeStruct((NTC, 8), jnp.float32),
    )(x, w1, b1, w2, b2)


def kernel(x, edge_index, edge_attr, params):
    p = params
    f32 = jnp.float32

    x8 = jnp.zeros((NTC, 8), f32).at[:N, :5].set(x)
    src = edge_index[0]
    dst = edge_index[1]
    padg = EP - E
    zpad = jnp.zeros((padg,), jnp.int32)
    ia = jnp.concatenate([dst, zpad]).reshape(2, NC * NS, WCHUNKS, 128)
    ib = jnp.concatenate([src, zpad]).reshape(2, NC * NS, WCHUNKS, 128)
    dst_s = jnp.concatenate(
        [dst, jnp.full((padg,), N, jnp.int32)]).reshape(2, NS, 4, 49, 128)
    ea8 = jnp.zeros((EP, 8), f32).at[:E, :4].set(edge_attr)
    eap = ea8.reshape(GE, 16, 64, 8).transpose(0, 2, 1, 3).reshape(
        2, EH // 16, 128)
    zrows = jnp.zeros((ROWS_PT, 32), f32)

    iw1 = jnp.zeros((8, H), f32).at[:5].set(p['iW1'])
    xc = _input_mlp(x8, iw1, p['ib1'][None], p['iln1_g'][None],
                    p['iln1_b'][None], p['iW2'], p['ib2'][None],
                    p['iln2_g'][None], p['iln2_b'][None])

    for l in range(3):
        xa, xb = _pre_mlp(xc, p['eW1'][l][:H], p['eW1'][l][H:2 * H],
                          p['eb1'][l][None])
        wc = jnp.zeros((8, 2 * H), f32).at[:4].set(p['eW1'][l][2 * H:])
        ew = (wc, p['eln1_g'][l][None], p['eln1_b'][l][None],
              p['eW2'][l], p['eb2'][l][None],
              p['eln2_g'][l][None], p['eln2_b'][l][None])
        m0 = _gather_sc(xa, xb, ia[0], ib[0])
        m1 = _gather_sc(xa, xb, ia[1], ib[1])
        e20 = _edge_mlp(m0, eap[0], *ew)
        e21 = _edge_mlp(m1, eap[1], *ew)
        agg0 = _scatter_sc(e20, dst_s[0], zrows)
        agg1 = _scatter_sc(e21, dst_s[1], zrows)
        xc = _node_mlp(xc, agg0, agg1,
                       p['nW1'][l][:H], p['nW1'][l][H:H + 32],
                       p['nW1'][l][H + 32:], p['nb1'][l][None],
                       p['nln1_g'][l][None], p['nln1_b'][l][None],
                       p['nW2'][l], p['nb2'][l][None],
                       p['nln2_g'][l][None], p['nln2_b'][l][None])

    ow2 = jnp.zeros((H, 8), f32).at[:, :1].set(p['oW2'])
    ob2 = jnp.zeros((1, 8), f32).at[0, 0].set(p['ob2'][0])
    out8 = _out_mlp(xc, p['oW1'], p['ob1'][None], ow2, ob2)
    return out8[:N, 0]


# speedup vs baseline: 1.3028x; 1.0989x over previous
import functools

import jax
import jax.numpy as jnp
from jax import lax
from jax.experimental import pallas as pl
from jax.experimental.pallas import tpu as pltpu
from jax.experimental.pallas import tpu_sc as plsc

N = 50000
E = 800000
H = 64
NTC = 50176
EP = 802816
NACC = 51200
TEDGE = 1024
TNODE = 1024
GE = EP // TEDGE
GN = NTC // TNODE
NC, NS = 2, 16
EH = EP // 4
GEH = EH // TEDGE
EPT = EH // NS
CHUNKS = EPT // 128
EPW = EH // (NC * NS)
WCHUNKS = EPW // 128
ROWS_PT = NACC // NS

_SQRT1_2 = 0.7071067811865476


def _gelu(t):
    return t * 0.5 * (1.0 + lax.erf(t * _SQRT1_2))


def _lnk(t, g, b):
    m = jnp.mean(t, axis=-1, keepdims=True)
    v = jnp.mean((t - m) ** 2, axis=-1, keepdims=True)
    return (t - m) * lax.rsqrt(v + 1e-5) * g + b


def _dot(a, b):
    return lax.dot_general(a, b, (((1,), (0,)), ((), ())),
                           preferred_element_type=jnp.float32)


def _wspec(shape):
    nd = len(shape)
    return pl.BlockSpec(shape, lambda i: (0,) * nd)


def _gather_body(xa_hbm, xb_hbm, ia_hbm, ib_hbm, m_hbm, ia_v, ib_v, rows_v,
                 sem):
    c = lax.axis_index("c")
    s = lax.axis_index("s")
    w = s * NC + c
    pltpu.sync_copy(ia_hbm.at[w], ia_v)
    pltpu.sync_copy(ib_hbm.at[w], ib_v)
    base = w * EPW

    def body(j, carry):
        pltpu.async_copy(xa_hbm.at[ia_v.at[j]], rows_v, sem).wait()
        pltpu.async_copy(xb_hbm.at[ib_v.at[j]], rows_v, sem, add=True).wait()
        pltpu.sync_copy(rows_v, m_hbm.at[pl.ds(base + j * 128, 128)])
        return carry

    lax.fori_loop(0, WCHUNKS, body, 0)


def _scatter_body(e_hbm, dst_hbm, z_hbm, out_hbm, idx_v, ebuf, acc):
    c = lax.axis_index("c")
    s = lax.axis_index("s")
    pltpu.sync_copy(z_hbm, acc.at[pl.ds(s * ROWS_PT, ROWS_PT)])
    plsc.subcore_barrier()
    base = s * EPT

    def outer(g, carry):
        pltpu.sync_copy(dst_hbm.at[s, g], idx_v)

        def body(j, carry2):
            pltpu.sync_copy(
                e_hbm.at[c, pl.ds(base + (g * 49 + j) * 128, 128)], ebuf)
            pltpu.sync_copy(ebuf, acc.at[idx_v.at[j]], add=True)
            return carry2

        lax.fori_loop(0, 49, body, carry)
        return carry

    lax.fori_loop(0, 2, outer, 0)
    plsc.subcore_barrier()
    pltpu.sync_copy(acc.at[pl.ds(s * ROWS_PT, ROWS_PT)],
                    out_hbm.at[c, pl.ds(s * ROWS_PT, ROWS_PT)])


@functools.cache
def _sc_kernels():
    mesh = plsc.VectorSubcoreMesh(core_axis_name="c", subcore_axis_name="s",
                                  num_cores=NC, num_subcores=NS)
    gather = pl.kernel(
        _gather_body,
        out_type=jax.ShapeDtypeStruct((EH, 2 * H), jnp.float32),
        mesh=mesh,
        scratch_types=[
            pltpu.VMEM((WCHUNKS, 128), jnp.int32),
            pltpu.VMEM((WCHUNKS, 128), jnp.int32),
            pltpu.VMEM((128, 2 * H), jnp.float32),
            pltpu.SemaphoreType.DMA,
        ],
    )
    scatter = pl.kernel(
        _scatter_body,
        out_type=jax.ShapeDtypeStruct((2, NACC, 32), jnp.float32),
        mesh=mesh,
        scratch_types=[
            pltpu.VMEM((49, 128), jnp.int32),
            pltpu.VMEM((128, 32), jnp.float32),
            pltpu.VMEM_SHARED((NACC, 32), jnp.float32),
        ],
        compiler_params=pltpu.CompilerParams(use_tc_tiling_on_sc=False),
    )
    return gather, scatter


def _gather_sc(xa, xb, ia, ib):
    return _sc_kernels()[0](xa, xb, ia, ib)


def _scatter_sc(e2, dst_s, zrows):
    return _sc_kernels()[1](e2, dst_s, zrows)


def _input_mlp(x8, w1, b1, g1, e1, w2, b2, g2, e2):
    def body(x_ref, w1r, b1r, g1r, e1r, w2r, b2r, g2r, e2r, o_ref):
        h = _lnk(_dot(x_ref[...], w1r[...]) + b1r[...], g1r[...], e1r[...])
        h = _gelu(h)
        o_ref[...] = _lnk(_dot(h, w2r[...]) + b2r[...], g2r[...], e2r[...])

    return pl.pallas_call(
        body,
        grid=(GN,),
        in_specs=[pl.BlockSpec((TNODE, 8), lambda i: (i, 0)),
                  _wspec((8, H)), _wspec((1, H)), _wspec((1, H)), _wspec((1, H)),
                  _wspec((H, H)), _wspec((1, H)), _wspec((1, H)), _wspec((1, H))],
        out_specs=pl.BlockSpec((TNODE, H), lambda i: (i, 0)),
        out_shape=jax.ShapeDtypeStruct((NTC, H), jnp.float32),
    )(x8, w1, b1, g1, e1, w2, b2, g2, e2)


def _pre_mlp(x, wa, wb, b1):
    def body(x_ref, war, wbr, b1r, oa_ref, ob_ref):
        xv = x_ref[...]
        oa_ref[...] = (_dot(xv, war[...]) + b1r[...])
        ob_ref[...] = _dot(xv, wbr[...])

    return pl.pallas_call(
        body,
        grid=(GN,),
        in_specs=[pl.BlockSpec((TNODE, H), lambda i: (i, 0)),
                  _wspec((H, 2 * H)), _wspec((H, 2 * H)), _wspec((1, 2 * H))],
        out_specs=[pl.BlockSpec((TNODE, 2 * H), lambda i: (i, 0)),
                   pl.BlockSpec((TNODE, 2 * H), lambda i: (i, 0))],
        out_shape=[jax.ShapeDtypeStruct((NTC, 2 * H), jnp.float32),
                   jax.ShapeDtypeStruct((NTC, 2 * H), jnp.float32)],
    )(x, wa, wb, b1)


def _edge_mlp(m, eap, wc, g1, e1, w2, b2, g2, e2):
    def body(m_ref, ea_ref, wcr, g1r, e1r, w2r, b2r, g2r, e2r, o_ref):
        eab = ea_ref[...]
        ea = jnp.concatenate([eab[:, 8 * k:8 * (k + 1)] for k in range(16)],
                             axis=0)
        mm = m_ref[...] + _dot(ea, wcr[...])
        e = mm[:, :H] * mm[:, H:]
        e = _gelu(_lnk(e, g1r[...], e1r[...]))
        e = _dot(e, w2r[...]) + b2r[...]
        e = _gelu(_lnk(e, g2r[...], e2r[...]))
        o_ref[0] = e[:, :32]
        o_ref[1] = e[:, 32:]

    return pl.pallas_call(
        body,
        grid=(GEH,),
        in_specs=[pl.BlockSpec((TEDGE, 2 * H), lambda i: (i, 0)),
                  pl.BlockSpec((TEDGE // 16, 128), lambda i: (i, 0)),
                  _wspec((8, 2 * H)), _wspec((1, H)), _wspec((1, H)),
                  _wspec((H, H)), _wspec((1, H)), _wspec((1, H)), _wspec((1, H))],
        out_specs=pl.BlockSpec((2, TEDGE, 32), lambda i: (0, i, 0)),
        out_shape=jax.ShapeDtypeStruct((2, EH, 32), jnp.float32),
    )(m, eap, wc, g1, e1, w2, b2, g2, e2)


def _node_mlp(x, agga, aggb, wx, wa0, wa1, b1, g1, e1, w2, b2, g2, e2):
    def body(x_ref, a_ref, b_ref, wxr, wa0r, wa1r, b1r, g1r, e1r, w2r, b2r,
             g2r, e2r, o_ref):
        xv = x_ref[...]
        a0 = a_ref[0] + b_ref[0]
        a1 = a_ref[1] + b_ref[1]
        u = (_dot(xv, wxr[...]) + _dot(a0, wa0r[...])
             + _dot(a1, wa1r[...]) + b1r[...])
        u = _gelu(_lnk(u, g1r[...], e1r[...]))
        u = _lnk(_dot(u, w2r[...]) + b2r[...], g2r[...], e2r[...])
        o_ref[...] = _gelu(xv + u)

    return pl.pallas_call(
        body,
        grid=(GN,),
        in_specs=[pl.BlockSpec((TNODE, H), lambda i: (i, 0)),
                  pl.BlockSpec((2, TNODE, 32), lambda i: (0, i, 0)),
                  pl.BlockSpec((2, TNODE, 32), lambda i: (0, i, 0)),
                  _wspec((H, H)), _wspec((32, H)), _wspec((32, H)),
                  _wspec((1, H)), _wspec((1, H)), _wspec((1, H)),
                  _wspec((H, H)), _wspec((1, H)), _wspec((1, H)), _wspec((1, H))],
        out_specs=pl.BlockSpec((TNODE, H), lambda i: (i, 0)),
        out_shape=jax.ShapeDtypeStruct((NTC, H), jnp.float32),
    )(x, agga, aggb, wx, wa0, wa1, b1, g1, e1, w2, b2, g2, e2)


def _out_mlp(x, w1, b1, w2, b2):
    def body(x_ref, w1r, b1r, w2r, b2r, o_ref):
        h = _gelu(_dot(x_ref[...], w1r[...]) + b1r[...])
        o_ref[...] = _dot(h, w2r[...]) + b2r[...]

    return pl.pallas_call(
        body,
        grid=(GN,),
        in_specs=[pl.BlockSpec((TNODE, H), lambda i: (i, 0)),
                  _wspec((H, H)), _wspec((1, H)), _wspec((H, 8)), _wspec((1, 8))],
        out_specs=pl.BlockSpec((TNODE, 8), lambda i: (i, 0)),
        out_shape=jax.ShapeDtypeStruct((NTC, 8), jnp.float32),
    )(x, w1, b1, w2, b2)


def _sum_pair(a, b):
    def body(a_ref, b_ref, o_ref):
        o_ref[...] = a_ref[...] + b_ref[...]

    return pl.pallas_call(
        body,
        grid=(NACC // 2048,),
        in_specs=[pl.BlockSpec((2, 2048, 32), lambda i: (0, i, 0)),
                  pl.BlockSpec((2, 2048, 32), lambda i: (0, i, 0))],
        out_specs=pl.BlockSpec((2, 2048, 32), lambda i: (0, i, 0)),
        out_shape=jax.ShapeDtypeStruct((2, NACC, 32), jnp.float32),
    )(a, b)


def kernel(x, edge_index, edge_attr, params):
    p = params
    f32 = jnp.float32

    x8 = jnp.zeros((NTC, 8), f32).at[:N, :5].set(x)
    src = edge_index[0]
    dst = edge_index[1]
    padg = EP - E
    zpad = jnp.zeros((padg,), jnp.int32)
    ia = jnp.concatenate([dst, zpad]).reshape(4, NC * NS, WCHUNKS, 128)
    ib = jnp.concatenate([src, zpad]).reshape(4, NC * NS, WCHUNKS, 128)
    dst_s = jnp.concatenate(
        [dst, jnp.full((padg,), N, jnp.int32)]).reshape(4, NS, 2, 49, 128)
    ea8 = jnp.zeros((EP, 8), f32).at[:E, :4].set(edge_attr)
    eap = ea8.reshape(GE, 16, 64, 8).transpose(0, 2, 1, 3).reshape(
        4, EH // 16, 128)
    zrows = jnp.zeros((ROWS_PT, 32), f32)

    iw1 = jnp.zeros((8, H), f32).at[:5].set(p['iW1'])
    xc = _input_mlp(x8, iw1, p['ib1'][None], p['iln1_g'][None],
                    p['iln1_b'][None], p['iW2'], p['ib2'][None],
                    p['iln2_g'][None], p['iln2_b'][None])

    for l in range(3):
        xa, xb = _pre_mlp(xc, p['eW1'][l][:H], p['eW1'][l][H:2 * H],
                          p['eb1'][l][None])
        wc = jnp.zeros((8, 2 * H), f32).at[:4].set(p['eW1'][l][2 * H:])
        ew = (wc, p['eln1_g'][l][None], p['eln1_b'][l][None],
              p['eW2'][l], p['eb2'][l][None],
              p['eln2_g'][l][None], p['eln2_b'][l][None])
        ms = [_gather_sc(xa, xb, ia[q], ib[q]) for q in range(4)]
        es = [_edge_mlp(ms[q], eap[q], *ew) for q in range(4)]
        ags = [_scatter_sc(es[q], dst_s[q], zrows) for q in range(4)]
        agg0 = _sum_pair(ags[0], ags[1])
        agg1 = _sum_pair(ags[2], ags[3])
        xc = _node_mlp(xc, agg0, agg1,
                       p['nW1'][l][:H], p['nW1'][l][H:H + 32],
                       p['nW1'][l][H + 32:], p['nb1'][l][None],
                       p['nln1_g'][l][None], p['nln1_b'][l][None],
                       p['nW2'][l], p['nb2'][l][None],
                       p['nln2_g'][l][None], p['nln2_b'][l][None])

    ow2 = jnp.zeros((H, 8), f32).at[:, :1].set(p['oW2'])
    ob2 = jnp.zeros((1, 8), f32).at[0, 0].set(p['ob2'][0])
    out8 = _out_mlp(xc, p['oW1'], p['ob1'][None], ow2, ob2)
    return out8[:N, 0]
